# baseline (speedup 1.0000x reference)
import functools

import jax
import jax.numpy as jnp
from jax import lax
from jax.experimental import pallas as pl
from jax.experimental.pallas import tpu as pltpu
from jax.experimental.pallas import tpu_sc as plsc

N = 10000
E = 160000
D = 128
NB = 4

NC = 2
NS = 16
NW = NC * NS
CHUNK = 128
NCHUNK = E // CHUNK
FULL_T = NCHUNK // NW
ZROWS = 128
BN = 2000
N_PAD = 10112
ROWS_PER_SUB = N_PAD // NS
P1OFF = 12000


def _sc_segment_sum(edge_attr, edge_index):
    mesh = plsc.VectorSubcoreMesh(core_axis_name="c", subcore_axis_name="s")

    @functools.partial(
        pl.kernel,
        out_type=jax.ShapeDtypeStruct((P1OFF + N_PAD, D), jnp.float32),
        mesh=mesh,
        scratch_types=[
            pltpu.VMEM((2, 2, CHUNK), jnp.int32),
            pltpu.VMEM((2, CHUNK, D), jnp.float32),
            pltpu.VMEM((ZROWS, D), jnp.float32),
            pltpu.VMEM_SHARED((N_PAD, D), jnp.float32),
            pltpu.SemaphoreType.DMA,
            pltpu.SemaphoreType.DMA,
            pltpu.SemaphoreType.DMA,
        ],
    )
    def seg(edge_hbm, ei_hbm, out_hbm, idx_v, rows_v, zbuf, acc,
            f0, f1, zsem):
        c = lax.axis_index("c")
        s = lax.axis_index("s")
        w = c * NS + s
        rbase = s * ROWS_PER_SUB
        fsems = (f0, f1)

        def start(t, b):
            ch = t * NW + w
            pltpu.async_copy(ei_hbm.at[:, pl.ds(ch * CHUNK, CHUNK)],
                             idx_v.at[b], fsems[b])
            pltpu.async_copy(edge_hbm.at[pl.ds(ch * CHUNK, CHUNK)],
                             rows_v.at[b], fsems[b])

        def wait_fetch(b):
            pltpu.make_async_copy(ei_hbm.at[:, pl.ds(0, CHUNK)],
                                  idx_v.at[b], fsems[b]).wait()
            pltpu.make_async_copy(edge_hbm.at[pl.ds(0, CHUNK)],
                                  rows_v.at[b], fsems[b]).wait()

        start(0, 0)
        start(1, 1)

        def zrow(r, carry):
            for k in range(D // 16):
                zbuf[r, pl.ds(k * 16, 16)] = jnp.zeros((16,), jnp.float32)
            return carry

        lax.fori_loop(0, ZROWS, zrow, 0)
        nfull = ROWS_PER_SUB // ZROWS
        for k in range(nfull):
            pltpu.async_copy(zbuf, acc.at[pl.ds(rbase + k * ZROWS, ZROWS)], zsem)
        rem = ROWS_PER_SUB % ZROWS
        if rem:
            pltpu.async_copy(zbuf.at[pl.ds(0, rem)],
                             acc.at[pl.ds(rbase + ROWS_PER_SUB - rem, rem)], zsem)
        for k in range(nfull):
            pltpu.make_async_copy(zbuf, acc.at[pl.ds(rbase, ZROWS)], zsem).wait()
        if rem:
            pltpu.make_async_copy(zbuf.at[pl.ds(0, rem)],
                                  acc.at[pl.ds(rbase, rem)], zsem).wait()
        plsc.subcore_barrier()

        def ring(g, carry):
            for b in range(2):
                t = 2 * g + b
                wait_fetch(b)
                pltpu.sync_copy(rows_v.at[b], acc.at[idx_v.at[b, 1]], add=True)

                @pl.when(t + 2 < FULL_T)
                def _():
                    start(t + 2, b)
            return carry

        lax.fori_loop(0, (FULL_T - 1) // 2, ring, 0)

        nleft = NCHUNK - FULL_T * NW

        @pl.when(w < nleft)
        def _():
            start(FULL_T, 1)

        wait_fetch(0)
        pltpu.sync_copy(rows_v.at[0], acc.at[idx_v.at[0, 1]], add=True)

        @pl.when(w < nleft)
        def _():
            wait_fetch(1)
            pltpu.sync_copy(rows_v.at[1], acc.at[idx_v.at[1, 1]], add=True)

        plsc.subcore_barrier()
        pltpu.sync_copy(acc.at[pl.ds(rbase, ROWS_PER_SUB)],
                        out_hbm.at[pl.ds(c * P1OFF + rbase, ROWS_PER_SUB)])

    return seg(edge_attr, edge_index)


def _tc_body(x_ref, p0_ref, p1_ref, w1a_ref, w1bcat_ref, b1_ref, w2_ref,
             b2_ref, g_ref, bt_ref, o_ref):
    xn = x_ref[...]
    agg = p0_ref[...] + p1_ref[...]
    agg4 = (jnp.dot(agg.astype(jnp.bfloat16), w1bcat_ref[...],
                    preferred_element_type=jnp.float32)
            + b1_ref[...][None, :])
    for i in range(NB):
        u = (jnp.dot(xn.astype(jnp.bfloat16), w1a_ref[i],
                     preferred_element_type=jnp.float32)
             + agg4[:, i * D:(i + 1) * D])
        h = jnp.maximum(u, 0.0)
        v = (jnp.dot(h.astype(jnp.bfloat16), w2_ref[i],
                     preferred_element_type=jnp.float32)
             + b2_ref[i][None, :])
        mu = jnp.mean(v, axis=-1, keepdims=True)
        var = jnp.mean(v * v, axis=-1, keepdims=True) - mu * mu
        scale = lax.rsqrt(var + 1e-5) * g_ref[i][None, :]
        xn = (v - mu) * scale + bt_ref[i][None, :] + xn
    o_ref[...] = xn


def _tc_mlp(x, partials, w1a, w1bcat, b1cat, w2, nb2, ng, nbt):
    whole = lambda shape: pl.BlockSpec(shape, lambda i: (0,) * len(shape))
    return pl.pallas_call(
        _tc_body,
        grid=(N // BN,),
        in_specs=[
            pl.BlockSpec((BN, D), lambda i: (i, 0)),
            pl.BlockSpec((BN, D), lambda i: (i, 0)),
            pl.BlockSpec((BN, D), lambda i: (i + P1OFF // BN, 0)),
            whole((NB, D, D)),
            whole((D, NB * D)),
            whole((NB * D,)),
            whole((NB, D, D)),
            whole((NB, D)),
            whole((NB, D)),
            whole((NB, D)),
        ],
        out_specs=pl.BlockSpec((BN, D), lambda i: (i, 0)),
        out_shape=jax.ShapeDtypeStruct((N, D), jnp.float32),
    )(x, partials, partials, w1a, w1bcat, b1cat, w2, nb2, ng, nbt)


def kernel(x, edge_attr, pos, edge_index, eW1, eb1, eW2, eb2, eg, ebt,
           nW1, nb1, nW2, nb2, ng, nbt):
    partials = _sc_segment_sum(edge_attr, edge_index)
    w1a = nW1[:, :D, :].astype(jnp.bfloat16)
    w1bcat = nW1[:, D:, :].transpose(1, 0, 2).reshape(D, NB * D).astype(jnp.bfloat16)
    w2 = nW2.astype(jnp.bfloat16)
    b1cat = nb1.reshape(NB * D)
    return _tc_mlp(x, partials, w1a, w1bcat, b1cat, w2, nb2, ng, nbt)


# speedup vs baseline: 1.0032x; 1.0032x over previous
import functools

import jax
import jax.numpy as jnp
from jax import lax
from jax.experimental import pallas as pl
from jax.experimental.pallas import tpu as pltpu
from jax.experimental.pallas import tpu_sc as plsc

N = 10000
E = 160000
D = 128
NB = 4

NC = 2
NS = 16
NW = NC * NS
CHUNK = 128
NCHUNK = E // CHUNK
FULL_T = NCHUNK // NW
ZROWS = 40
BN = 2000
N_PAD = 10112
ROWS_PER_SUB = N_PAD // NS
P1OFF = 12000


def _sc_segment_sum(edge_attr, edge_index):
    mesh = plsc.VectorSubcoreMesh(core_axis_name="c", subcore_axis_name="s")

    @functools.partial(
        pl.kernel,
        out_type=jax.ShapeDtypeStruct((P1OFF + N_PAD, D), jnp.float32),
        mesh=mesh,
        scratch_types=[
            pltpu.VMEM((2, 2, CHUNK), jnp.int32),
            pltpu.VMEM((2, CHUNK, D), jnp.float32),
            pltpu.VMEM((ZROWS, D), jnp.float32),
            pltpu.VMEM_SHARED((N_PAD, D), jnp.float32),
            pltpu.SemaphoreType.DMA,
            pltpu.SemaphoreType.DMA,
            pltpu.SemaphoreType.DMA,
        ],
    )
    def seg(edge_hbm, ei_hbm, out_hbm, idx_v, rows_v, zbuf, acc,
            f0, f1, zsem):
        c = lax.axis_index("c")
        s = lax.axis_index("s")
        w = c * NS + s
        rbase = s * ROWS_PER_SUB
        fsems = (f0, f1)

        def start(t, b):
            ch = t * NW + w
            pltpu.async_copy(ei_hbm.at[:, pl.ds(ch * CHUNK, CHUNK)],
                             idx_v.at[b], fsems[b])
            pltpu.async_copy(edge_hbm.at[pl.ds(ch * CHUNK, CHUNK)],
                             rows_v.at[b], fsems[b])

        def wait_fetch(b):
            pltpu.make_async_copy(ei_hbm.at[:, pl.ds(0, CHUNK)],
                                  idx_v.at[b], fsems[b]).wait()
            pltpu.make_async_copy(edge_hbm.at[pl.ds(0, CHUNK)],
                                  rows_v.at[b], fsems[b]).wait()

        start(0, 0)
        start(1, 1)

        def zrow(r, carry):
            for k in range(D // 16):
                zbuf[r, pl.ds(k * 16, 16)] = jnp.zeros((16,), jnp.float32)
            return carry

        lax.fori_loop(0, ZROWS, zrow, 0)
        nfull = ROWS_PER_SUB // ZROWS
        for k in range(nfull):
            pltpu.async_copy(zbuf, acc.at[pl.ds(rbase + k * ZROWS, ZROWS)], zsem)
        rem = ROWS_PER_SUB % ZROWS
        if rem:
            pltpu.async_copy(zbuf.at[pl.ds(0, rem)],
                             acc.at[pl.ds(rbase + ROWS_PER_SUB - rem, rem)], zsem)
        for k in range(nfull):
            pltpu.make_async_copy(zbuf, acc.at[pl.ds(rbase, ZROWS)], zsem).wait()
        if rem:
            pltpu.make_async_copy(zbuf.at[pl.ds(0, rem)],
                                  acc.at[pl.ds(rbase, rem)], zsem).wait()
        plsc.subcore_barrier()

        def ring(g, carry):
            for b in range(2):
                t = 2 * g + b
                wait_fetch(b)
                pltpu.sync_copy(rows_v.at[b], acc.at[idx_v.at[b, 1]], add=True)

                @pl.when(t + 2 < FULL_T)
                def _():
                    start(t + 2, b)
            return carry

        lax.fori_loop(0, (FULL_T - 1) // 2, ring, 0)

        nleft = NCHUNK - FULL_T * NW

        @pl.when(w < nleft)
        def _():
            start(FULL_T, 1)

        wait_fetch(0)
        pltpu.sync_copy(rows_v.at[0], acc.at[idx_v.at[0, 1]], add=True)

        @pl.when(w < nleft)
        def _():
            wait_fetch(1)
            pltpu.sync_copy(rows_v.at[1], acc.at[idx_v.at[1, 1]], add=True)

        plsc.subcore_barrier()
        pltpu.sync_copy(acc.at[pl.ds(rbase, ROWS_PER_SUB)],
                        out_hbm.at[pl.ds(c * P1OFF + rbase, ROWS_PER_SUB)])

    return seg(edge_attr, edge_index)


def _tc_body(x_ref, p0_ref, p1_ref, w1a_ref, w1bcat_ref, b1_ref, w2_ref,
             b2_ref, g_ref, bt_ref, o_ref):
    xn = x_ref[...]
    agg = p0_ref[...] + p1_ref[...]
    agg4 = (jnp.dot(agg.astype(jnp.bfloat16), w1bcat_ref[...],
                    preferred_element_type=jnp.float32)
            + b1_ref[...][None, :])
    for i in range(NB):
        u = (jnp.dot(xn.astype(jnp.bfloat16), w1a_ref[i],
                     preferred_element_type=jnp.float32)
             + agg4[:, i * D:(i + 1) * D])
        h = jnp.maximum(u, 0.0)
        v = (jnp.dot(h.astype(jnp.bfloat16), w2_ref[i],
                     preferred_element_type=jnp.float32)
             + b2_ref[i][None, :])
        mu = jnp.mean(v, axis=-1, keepdims=True)
        var = jnp.mean(v * v, axis=-1, keepdims=True) - mu * mu
        scale = lax.rsqrt(var + 1e-5) * g_ref[i][None, :]
        xn = (v - mu) * scale + bt_ref[i][None, :] + xn
    o_ref[...] = xn


def _tc_mlp(x, partials, w1a, w1bcat, b1cat, w2, nb2, ng, nbt):
    whole = lambda shape: pl.BlockSpec(shape, lambda i: (0,) * len(shape))
    return pl.pallas_call(
        _tc_body,
        grid=(N // BN,),
        in_specs=[
            pl.BlockSpec((BN, D), lambda i: (i, 0)),
            pl.BlockSpec((BN, D), lambda i: (i, 0)),
            pl.BlockSpec((BN, D), lambda i: (i + P1OFF // BN, 0)),
            whole((NB, D, D)),
            whole((D, NB * D)),
            whole((NB * D,)),
            whole((NB, D, D)),
            whole((NB, D)),
            whole((NB, D)),
            whole((NB, D)),
        ],
        out_specs=pl.BlockSpec((BN, D), lambda i: (i, 0)),
        out_shape=jax.ShapeDtypeStruct((N, D), jnp.float32),
    )(x, partials, partials, w1a, w1bcat, b1cat, w2, nb2, ng, nbt)


def kernel(x, edge_attr, pos, edge_index, eW1, eb1, eW2, eb2, eg, ebt,
           nW1, nb1, nW2, nb2, ng, nbt):
    partials = _sc_segment_sum(edge_attr, edge_index)
    w1a = nW1[:, :D, :].astype(jnp.bfloat16)
    w1bcat = nW1[:, D:, :].transpose(1, 0, 2).reshape(D, NB * D).astype(jnp.bfloat16)
    w2 = nW2.astype(jnp.bfloat16)
    b1cat = nb1.reshape(NB * D)
    return _tc_mlp(x, partials, w1a, w1bcat, b1cat, w2, nb2, ng, nbt)
